# Initial kernel scaffold; baseline (speedup 1.0000x reference)
import jax
import jax.numpy as jnp
from jax.experimental import pallas as pl


def kernel(preds_verts, targets_verts, preds_tex, targets_tex, nbs_idxs, nbs_weights):
    raise NotImplementedError("write your pallas kernel here")


# speedup vs baseline: 13.8547x; 13.8547x over previous
import functools

import jax
import jax.numpy as jnp
from jax import lax
from jax.experimental import pallas as pl
from jax.experimental.pallas import tpu as pltpu
from jax.experimental.pallas import tpu_sc as plsc

_GEOMETRY_REC = 1.0
_GEOMETRY_LAPLACIAN = 0.1
_TEX_REC = 1.0

_B, _N, _K = 16, 50000, 8
_C = _B * 3
_NW = 32
_VPW = 1568
_SUB = 112
_NSUB = _VPW // _SUB
_NPAD = _NW * _VPW


def _sc_vertex_losses(d_t, idx_r, wgt_r):
  mesh = plsc.VectorSubcoreMesh(core_axis_name="c", subcore_axis_name="s")

  @functools.partial(
      pl.kernel,
      mesh=mesh,
      out_type=jax.ShapeDtypeStruct((_NW, 2, 16), jnp.float32),
      scratch_types=[
          pltpu.VMEM((_K, _SUB), jnp.int32),
          pltpu.VMEM((_K, _SUB), jnp.float32),
          pltpu.VMEM((_SUB, _C), jnp.float32),
          pltpu.VMEM((_K, _SUB, _C), jnp.float32),
          pltpu.VMEM((2, 16), jnp.float32),
          pltpu.SemaphoreType.DMA,
      ],
      compiler_params=pltpu.CompilerParams(
          needs_layout_passes=False, use_tc_tiling_on_sc=False),
  )
  def body(d_hbm, idx_hbm, wgt_hbm, out_hbm, idx_v, wgt_v, drow_v, gat_v,
           out_v, sem):
    wid = lax.axis_index("s") * 2 + lax.axis_index("c")
    base = wid * _VPW
    lane = lax.iota(jnp.int32, 16)

    def sub_chunk(i, sums):
      off = base + i * _SUB
      pltpu.sync_copy(idx_hbm.at[wid, i], idx_v)
      pltpu.sync_copy(wgt_hbm.at[wid, i], wgt_v)
      pltpu.sync_copy(d_hbm.at[pl.ds(off, _SUB)], drow_v)
      copies = [
          pltpu.async_copy(d_hbm.at[idx_v.at[k]], gat_v.at[k], sem)
          for k in range(_K)
      ]
      for cp in copies:
        cp.wait()

      def group(g, sums):
        sum_d2, sum_l2 = sums
        rows = g * 16 + lane
        wk = [wgt_v[k, pl.ds(g * 16, 16)] for k in range(_K)]
        for c in range(_C):
          col = jnp.full((16,), c, jnp.int32)
          dv = plsc.load_gather(drow_v, [rows, col])
          acc = dv
          for k in range(_K):
            gv = plsc.load_gather(gat_v.at[k], [rows, col])
            acc = acc + wk[k] * gv
          sum_d2 = sum_d2 + dv * dv
          sum_l2 = sum_l2 + acc * acc
        return (sum_d2, sum_l2)

      return lax.fori_loop(0, _SUB // 16, group, sums)

    zeros = jnp.zeros((16,), jnp.float32)
    sum_d2, sum_l2 = lax.fori_loop(0, _NSUB, sub_chunk, (zeros, zeros))
    out_v[0, :] = sum_d2
    out_v[1, :] = sum_l2
    pltpu.sync_copy(out_v, out_hbm.at[wid])

  return body(d_t, idx_r, wgt_r)


def _tex_l1_body(p_ref, t_ref, o_ref):
  i = pl.program_id(0)
  s = jnp.sum(jnp.abs(p_ref[...] - t_ref[...]))

  @pl.when(i == 0)
  def _():
    o_ref[0, 0] = s

  @pl.when(i > 0)
  def _():
    o_ref[0, 0] += s


def _tex_l1_sum(p2d, t2d):
  rows, cols = p2d.shape
  block_rows = 512
  grid = rows // block_rows
  return pl.pallas_call(
      _tex_l1_body,
      grid=(grid,),
      in_specs=[
          pl.BlockSpec((block_rows, cols), lambda i: (i, 0)),
          pl.BlockSpec((block_rows, cols), lambda i: (i, 0)),
      ],
      out_specs=pl.BlockSpec((1, 1), lambda i: (0, 0),
                             memory_space=pltpu.SMEM),
      out_shape=jax.ShapeDtypeStruct((1, 1), jnp.float32),
      compiler_params=pltpu.CompilerParams(
          dimension_semantics=("arbitrary",)),
  )(p2d, t2d)


def kernel(preds_verts, targets_verts, preds_tex, targets_tex, nbs_idxs,
           nbs_weights):
  d = preds_verts - targets_verts
  d_t = jnp.transpose(d, (1, 0, 2)).reshape(_N, _C)
  d_t = jnp.pad(d_t, ((0, _NPAD - _N), (0, 0)))

  idx_t = jnp.pad(nbs_idxs.astype(jnp.int32).T, ((0, 0), (0, _NPAD - _N)))
  wgt_t = jnp.pad(nbs_weights.T, ((0, 0), (0, _NPAD - _N)))
  idx_r = idx_t.reshape(_K, _NW, _NSUB, _SUB).transpose(1, 2, 0, 3)
  wgt_r = wgt_t.reshape(_K, _NW, _NSUB, _SUB).transpose(1, 2, 0, 3)

  vert_partials = _sc_vertex_losses(d_t, idx_r, wgt_r)

  tex_elems = preds_tex.size
  p2d = preds_tex.reshape(tex_elems // 1024, 1024)
  t2d = targets_tex.reshape(tex_elems // 1024, 1024)
  tex_sum = _tex_l1_sum(p2d, t2d)[0, 0]

  denom_verts = jnp.float32(_B * _N * 3)
  loss_verts_rec = jnp.sum(vert_partials[:, 0, :]) / denom_verts
  loss_verts_laplacian = jnp.sum(vert_partials[:, 1, :]) / denom_verts
  loss_tex_rec = tex_sum / jnp.float32(tex_elems)
  loss = (loss_verts_rec * _GEOMETRY_REC
          + loss_verts_laplacian * _GEOMETRY_LAPLACIAN
          + loss_tex_rec * _TEX_REC)
  return (loss, loss_verts_rec, loss_verts_laplacian, loss_tex_rec)
